# baseline (speedup 1.0000x reference)
import jax
import jax.numpy as jnp
from jax import lax
from jax.experimental import pallas as pl
from jax.experimental.pallas import tpu as pltpu
from jax.experimental.pallas import tpu_sc as plsc

ROWS = 1_000_000
D = 64
NIDX = 16384

NC = 2
NS = 16
NW = NC * NS
RPT = ROWS // NW
C = 625
CPT = RPT // C
NCH = NW * CPT
PREC = 16


def _lane(vec, j):
    return lax.squeeze(lax.slice(vec, [j], [j + 1]), [0])


def _sc_body(sidx_hbm, order_hbm, pairs_hbm, a_hbm, b_hbm, out_hbm,
             abuf, bbuf, sbuf, obuf, stbuf, sem):
    wid = lax.axis_index("s") * NC + lax.axis_index("c")

    def chunk_body(kk, carry):
        base_row = wid * RPT + kk * C
        gk = wid * CPT + kk
        pltpu.sync_copy(pairs_hbm.at[pl.ds(gk * PREC, PREC)], stbuf)
        pltpu.sync_copy(a_hbm.at[pl.ds(base_row * D, C * D)], abuf)
        sv = stbuf[...]
        s = _lane(sv, 0)
        e = _lane(sv, 1)

        def batch_body(b, bcarry):
            bb = b * 16
            pltpu.sync_copy(sidx_hbm.at[pl.ds(bb, 16)], sbuf)
            pltpu.sync_copy(order_hbm.at[pl.ds(bb, 16)], obuf)
            pltpu.async_copy(b_hbm.at[obuf], bbuf, sem).wait()
            lv = sbuf[...] - base_row
            for j in range(16):
                pos = bb + j
                cond = jnp.logical_and(pos >= s, pos < e)

                @pl.when(cond)
                def _(j=j, lv=lv):
                    lj = _lane(lv, j)
                    for c in range(4):
                        x = bbuf[j, pl.ds(c * 16, 16)]
                        plsc.addupdate(abuf.at[pl.ds(lj * D + c * 16, 16)], x)
            return bcarry

        lax.fori_loop(s // 16, (e + 15) // 16, batch_body, 0)
        pltpu.sync_copy(abuf, out_hbm.at[pl.ds(base_row * D, C * D)])
        return carry

    lax.fori_loop(0, CPT, chunk_body, 0)


@jax.jit
def _scatter_add(sidx, order, pairs, A_flat, B):
    mesh = plsc.VectorSubcoreMesh(
        core_axis_name="c", subcore_axis_name="s",
        num_cores=NC, num_subcores=NS)
    f = pl.kernel(
        _sc_body,
        out_type=jax.ShapeDtypeStruct((ROWS * D,), jnp.float32),
        mesh=mesh,
        compiler_params=pltpu.CompilerParams(use_tc_tiling_on_sc=False),
        scratch_types=[
            pltpu.VMEM((C * D,), jnp.float32),
            pltpu.VMEM((16, D), jnp.float32),
            pltpu.VMEM((16,), jnp.int32),
            pltpu.VMEM((16,), jnp.int32),
            pltpu.VMEM((PREC,), jnp.int32),
            pltpu.SemaphoreType.DMA,
        ],
    )
    return f(sidx, order, pairs, A_flat, B)


def kernel(index, A, B):
    index = index.astype(jnp.int32)
    order = jnp.argsort(index).astype(jnp.int32)
    sidx = index[order]
    bounds = jnp.arange(0, NCH + 1, dtype=jnp.int32) * C
    starts = jnp.searchsorted(sidx, bounds).astype(jnp.int32)
    pairs = jnp.stack([starts[:NCH], starts[1:]], axis=-1)
    pairs = jnp.pad(pairs, ((0, 0), (0, PREC - 2))).reshape(-1)
    out = _scatter_add(sidx, order, pairs, A.reshape(-1), B)
    return out.reshape(ROWS, D)


# speedup vs baseline: 1.1178x; 1.1178x over previous
import jax
import jax.numpy as jnp
from jax import lax
from jax.experimental import pallas as pl
from jax.experimental.pallas import tpu as pltpu
from jax.experimental.pallas import tpu_sc as plsc

ROWS = 1_000_000
D = 64
NIDX = 16384

NC = 2
NS = 16
NW = NC * NS
RPT = ROWS // NW
C = 625
CPT = RPT // C
NCH = NW * CPT
PREC = 16


def _lane(vec, j):
    return lax.squeeze(lax.slice(vec, [j], [j + 1]), [0])


def _sc_body(sidx_hbm, order_hbm, pairs_hbm, a_hbm, b_hbm, out_hbm,
             ab0, ab1, bbuf, sxbuf, odbuf, stbuf,
             semL0, semL1, semS0, semS1, semB):
    wid = lax.axis_index("s") * NC + lax.axis_index("c")
    ab = (ab0, ab1)
    semL = (semL0, semL1)
    semS = (semS0, semS1)

    pltpu.sync_copy(sidx_hbm, sxbuf)
    pltpu.sync_copy(order_hbm, odbuf)
    pltpu.sync_copy(pairs_hbm.at[pl.ds(wid * CPT * PREC, CPT * PREC)], stbuf)

    def chunk_off(kk):
        return (wid * RPT + kk * C) * D

    pltpu.async_copy(a_hbm.at[pl.ds(chunk_off(0), C * D)], ab0, semL0)

    def process(kk, cur):
        base_row = wid * RPT + kk * C
        rec = stbuf[pl.ds(kk * PREC, 16)]
        s = _lane(rec, 0)
        e = _lane(rec, 1)

        def batch_body(b, bcarry):
            bb = b * 16
            sv = sxbuf[pl.ds(bb, 16)]
            ov = odbuf[pl.ds(bb, 16)]
            pltpu.async_copy(b_hbm.at[ov], bbuf, semB).wait()
            lv = sv - base_row
            for j in range(16):
                pos = bb + j
                cond = jnp.logical_and(pos >= s, pos < e)

                @pl.when(cond)
                def _(j=j, lv=lv):
                    lj = _lane(lv, j)
                    for c in range(4):
                        x = bbuf[j, pl.ds(c * 16, 16)]
                        plsc.addupdate(cur.at[pl.ds(lj * D + c * 16, 16)], x)
            return bcarry

        lax.fori_loop(s // 16, (e + 15) // 16, batch_body, 0)

    def pair_body(i, carry):
        for par in range(2):
            kk = 2 * i + par
            cur, nxt = ab[par], ab[1 - par]
            pltpu.make_async_copy(
                a_hbm.at[pl.ds(0, C * D)], cur, semL[par]).wait()
            if par == 0:
                @pl.when(i >= 1)
                def _():
                    pltpu.make_async_copy(
                        nxt, out_hbm.at[pl.ds(0, C * D)], semS[1]).wait()
                pltpu.async_copy(
                    a_hbm.at[pl.ds(chunk_off(kk + 1), C * D)], nxt, semL[1])
            else:
                pltpu.make_async_copy(
                    nxt, out_hbm.at[pl.ds(0, C * D)], semS[0]).wait()

                @pl.when(i < CPT // 2 - 1)
                def _(kk=kk):
                    pltpu.async_copy(
                        a_hbm.at[pl.ds(chunk_off(kk + 1), C * D)],
                        nxt, semL[0])
            process(kk, cur)
            pltpu.async_copy(
                cur, out_hbm.at[pl.ds(chunk_off(kk), C * D)], semS[par])
        return carry

    lax.fori_loop(0, CPT // 2, pair_body, 0)
    pltpu.make_async_copy(ab1, out_hbm.at[pl.ds(0, C * D)], semS1).wait()


@jax.jit
def _scatter_add(sidx, order, pairs, A_flat, B):
    mesh = plsc.VectorSubcoreMesh(
        core_axis_name="c", subcore_axis_name="s",
        num_cores=NC, num_subcores=NS)
    f = pl.kernel(
        _sc_body,
        out_type=jax.ShapeDtypeStruct((ROWS * D,), jnp.float32),
        mesh=mesh,
        compiler_params=pltpu.CompilerParams(use_tc_tiling_on_sc=False),
        scratch_types=[
            pltpu.VMEM((C * D,), jnp.float32),
            pltpu.VMEM((C * D,), jnp.float32),
            pltpu.VMEM((16, D), jnp.float32),
            pltpu.VMEM((NIDX,), jnp.int32),
            pltpu.VMEM((NIDX,), jnp.int32),
            pltpu.VMEM((CPT * PREC,), jnp.int32),
            pltpu.SemaphoreType.DMA,
            pltpu.SemaphoreType.DMA,
            pltpu.SemaphoreType.DMA,
            pltpu.SemaphoreType.DMA,
            pltpu.SemaphoreType.DMA,
        ],
    )
    return f(sidx, order, pairs, A_flat, B)


def kernel(index, A, B):
    index = index.astype(jnp.int32)
    order = jnp.argsort(index).astype(jnp.int32)
    sidx = index[order]
    bounds = jnp.arange(0, NCH + 1, dtype=jnp.int32) * C
    starts = jnp.searchsorted(sidx, bounds).astype(jnp.int32)
    pairs = jnp.stack([starts[:NCH], starts[1:]], axis=-1)
    pairs = jnp.pad(pairs, ((0, 0), (0, PREC - 2))).reshape(-1)
    out = _scatter_add(sidx, order, pairs, A.reshape(-1), B)
    return out.reshape(ROWS, D)


# speedup vs baseline: 1.1558x; 1.0340x over previous
import jax
import jax.numpy as jnp
from jax import lax
from jax.experimental import pallas as pl
from jax.experimental.pallas import tpu as pltpu
from jax.experimental.pallas import tpu_sc as plsc

ROWS = 1_000_000
D = 64
NIDX = 16384

NC = 2
NS = 16
NW = NC * NS
C = 320
NCH = ROWS // C
KMAX = -(-NCH // NW)
PREC = 16


def _lane(vec, j):
    return lax.squeeze(lax.slice(vec, [j], [j + 1]), [0])


def _sc_body(sidx_hbm, order_hbm, pairs_hbm, a_hbm, b_hbm, out_hbm,
             ab0, ab1, bbuf, sxbuf, odbuf, stbuf,
             semL0, semL1, semS0, semS1, semB):
    wid = lax.axis_index("s") * NC + lax.axis_index("c")
    ab = (ab0, ab1)
    semL = (semL0, semL1)
    semS = (semS0, semS1)
    cnt = jnp.where(wid < NCH % NW, NCH // NW + 1, NCH // NW)

    pltpu.sync_copy(sidx_hbm, sxbuf)
    pltpu.sync_copy(order_hbm, odbuf)
    pltpu.sync_copy(
        pairs_hbm.at[pl.ds(wid * KMAX * PREC, KMAX * PREC)], stbuf)

    def row0(kk):
        return (wid + NW * kk) * C

    def start_load(kk, par):
        pltpu.async_copy(a_hbm.at[pl.ds(row0(kk), C)], ab[par], semL[par])

    def process(kk, cur):
        base_row = row0(kk)
        rec = stbuf[pl.ds(kk * PREC, 16)]
        s = _lane(rec, 0)
        e = _lane(rec, 1)

        def batch_body(b, bcarry):
            bb = b * 16
            sv = sxbuf[pl.ds(bb, 16)]
            ov = odbuf[pl.ds(bb, 16)]
            pltpu.async_copy(b_hbm.at[ov], bbuf, semB).wait()
            lv = sv - base_row
            for j in range(16):
                pos = bb + j
                cond = jnp.logical_and(pos >= s, pos < e)

                @pl.when(cond)
                def _(j=j, lv=lv):
                    lj = _lane(lv, j)
                    for c in range(4):
                        x = bbuf[j, pl.ds(c * 16, 16)]
                        plsc.addupdate(cur.at[lj, pl.ds(c * 16, 16)], x)
            return bcarry

        lax.fori_loop(s // 16, (e + 15) // 16, batch_body, 0)

    start_load(0, 0)

    def body(kk, carry):
        for par in range(2):
            @pl.when(kk % 2 == par)
            def _(par=par):
                cur, nxt = ab[par], ab[1 - par]
                pltpu.make_async_copy(
                    a_hbm.at[pl.ds(0, C)], cur, semL[par]).wait()

                @pl.when(kk + 1 < cnt)
                def _():
                    @pl.when(kk >= 1)
                    def _():
                        pltpu.make_async_copy(
                            nxt, out_hbm.at[pl.ds(0, C)],
                            semS[1 - par]).wait()
                    start_load(kk + 1, 1 - par)

                process(kk, cur)
                pltpu.async_copy(
                    cur, out_hbm.at[pl.ds(row0(kk), C)], semS[par])
        return carry

    lax.fori_loop(0, cnt, body, 0)
    pltpu.make_async_copy(ab0, out_hbm.at[pl.ds(0, C)], semS0).wait()
    pltpu.make_async_copy(ab1, out_hbm.at[pl.ds(0, C)], semS1).wait()


@jax.jit
def _scatter_add(sidx, order, pairs, A, B):
    mesh = plsc.VectorSubcoreMesh(
        core_axis_name="c", subcore_axis_name="s",
        num_cores=NC, num_subcores=NS)
    f = pl.kernel(
        _sc_body,
        out_type=jax.ShapeDtypeStruct((ROWS, D), jnp.float32),
        mesh=mesh,
        scratch_types=[
            pltpu.VMEM((C, D), jnp.float32),
            pltpu.VMEM((C, D), jnp.float32),
            pltpu.VMEM((16, 2 * D), jnp.float32),
            pltpu.VMEM((NIDX,), jnp.int32),
            pltpu.VMEM((NIDX,), jnp.int32),
            pltpu.VMEM((KMAX * PREC,), jnp.int32),
            pltpu.SemaphoreType.DMA,
            pltpu.SemaphoreType.DMA,
            pltpu.SemaphoreType.DMA,
            pltpu.SemaphoreType.DMA,
            pltpu.SemaphoreType.DMA,
        ],
    )
    return f(sidx, order, pairs, A, B)


def kernel(index, A, B):
    index = index.astype(jnp.int32)
    order = jnp.argsort(index).astype(jnp.int32)
    sidx = index[order]
    bounds = jnp.arange(0, NCH + 1, dtype=jnp.int32) * C
    starts = jnp.searchsorted(sidx, bounds).astype(jnp.int32)
    wids = jnp.arange(NW, dtype=jnp.int32)[:, None]
    kks = jnp.arange(KMAX, dtype=jnp.int32)[None, :]
    gk = wids + NW * kks
    valid = gk < NCH
    gkc = jnp.minimum(gk, NCH - 1)
    s = jnp.where(valid, starts[gkc], 0)
    e = jnp.where(valid, starts[gkc + 1], 0)
    rec = jnp.stack([s, e], axis=-1)
    rec = jnp.pad(rec, ((0, 0), (0, 0), (0, PREC - 2))).reshape(-1)
    B_pad = jnp.pad(B, ((0, 0), (0, D)))
    return _scatter_add(sidx, order, rec, A, B_pad)
